# baseline (speedup 1.0000x reference)
import functools

import jax
import jax.numpy as jnp
from jax import lax
from jax.experimental import pallas as pl
from jax.experimental.pallas import tpu as pltpu
from jax.experimental.pallas import tpu_sc as plsc

N_USERS = 1000000
N_ITEMS = 100000
N_RANK = 64
BATCH = 16384

NUM_CORES = 2
NUM_SUBCORES = 16
NUM_WORKERS = NUM_CORES * NUM_SUBCORES
N_PER_W = BATCH // NUM_WORKERS
LANES = 16
N_GROUPS = N_PER_W // LANES

SELU_ALPHA = 1.6732632423543772
SELU_SCALE = 1.0507009873554805

_mesh = plsc.VectorSubcoreMesh(core_axis_name="c", subcore_axis_name="s")


@functools.partial(
    pl.kernel,
    out_type=jax.ShapeDtypeStruct((BATCH,), jnp.float32),
    mesh=_mesh,
    compiler_params=pltpu.CompilerParams(
        needs_layout_passes=False, use_tc_tiling_on_sc=False),
    scratch_types=[
        pltpu.VMEM((N_PER_W,), jnp.int32),
        pltpu.VMEM((N_PER_W,), jnp.int32),
        pltpu.VMEM((N_PER_W, N_RANK), jnp.float32),
        pltpu.VMEM((N_PER_W, N_RANK), jnp.float32),
        pltpu.VMEM((N_PER_W,), jnp.float32),
        pltpu.VMEM((N_PER_W,), jnp.float32),
        pltpu.VMEM((N_PER_W,), jnp.float32),
        pltpu.VMEM((LANES,), jnp.float32),
        pltpu.SemaphoreType.DMA,
    ],
)
def _lfm_sc(uid_hbm, iid_hbm, p_hbm, q_hbm, mu_hbm, ub_hbm, ib_hbm,
            out_hbm, uid_v, iid_v, urows_v, irows_v, ub_v, ib_v, out_v,
            mu_sm, sem):
    wid = lax.axis_index("s") * NUM_CORES + lax.axis_index("c")
    base = wid * N_PER_W

    pltpu.sync_copy(uid_hbm.at[pl.ds(base, N_PER_W)], uid_v)
    pltpu.sync_copy(iid_hbm.at[pl.ds(base, N_PER_W)], iid_v)
    pltpu.sync_copy(mu_hbm, mu_sm)

    cp1 = pltpu.async_copy(p_hbm.at[uid_v], urows_v, sem)
    cp2 = pltpu.async_copy(q_hbm.at[iid_v], irows_v, sem)
    cp3 = pltpu.async_copy(ub_hbm.at[uid_v], ub_v, sem)
    cp4 = pltpu.async_copy(ib_hbm.at[iid_v], ib_v, sem)
    cp1.wait()
    cp2.wait()
    cp3.wait()
    cp4.wait()

    mu = mu_sm[...]
    lane = lax.iota(jnp.int32, LANES)

    @pl.loop(0, N_GROUPS)
    def _(g):
        rows = g * LANES + lane
        acc = jnp.zeros((LANES,), jnp.float32)
        for d in range(N_RANK):
            col = jnp.full((LANES,), d, jnp.int32)
            u = plsc.load_gather(urows_v, [rows, col])
            t = plsc.load_gather(irows_v, [rows, col])
            x = u * t
            acc = acc + jnp.where(x > 0.0, x, SELU_ALPHA * (jnp.exp(x) - 1.0))
        sl = pl.ds(g * LANES, LANES)
        out_v[sl] = SELU_SCALE * acc + ub_v[sl] + ib_v[sl] + mu

    pltpu.sync_copy(out_v, out_hbm.at[pl.ds(base, N_PER_W)])


def kernel(user_ids, item_ids, P, Q, mu, user_bias, item_bias):
    uid = user_ids.astype(jnp.int32)
    iid = item_ids.astype(jnp.int32)
    mu16 = jnp.broadcast_to(mu.astype(jnp.float32), (LANES,))
    return _lfm_sc(uid, iid, P, Q, mu16, user_bias, item_bias)


# speedup vs baseline: 1.6085x; 1.6085x over previous
import functools

import jax
import jax.numpy as jnp
from jax import lax
from jax.experimental import pallas as pl
from jax.experimental.pallas import tpu as pltpu
from jax.experimental.pallas import tpu_sc as plsc

N_RANK = 64
BATCH = 16384

NUM_CORES = 2
NUM_SUBCORES = 16
NUM_WORKERS = NUM_CORES * NUM_SUBCORES
N_PER_W = BATCH // NUM_WORKERS
LANES = 16
N_GROUPS = N_PER_W // LANES

SELU_ALPHA = 1.6732632423543772
SELU_SCALE = 1.0507009873554805

_mesh = plsc.VectorSubcoreMesh(core_axis_name="c", subcore_axis_name="s")


@functools.partial(
    pl.kernel,
    out_type=jax.ShapeDtypeStruct((BATCH,), jnp.float32),
    mesh=_mesh,
    compiler_params=pltpu.CompilerParams(
        needs_layout_passes=False, use_tc_tiling_on_sc=True),
    scratch_types=[
        pltpu.VMEM((N_PER_W,), jnp.int32),
        pltpu.VMEM((N_PER_W,), jnp.int32),
        pltpu.VMEM((N_PER_W, 2 * N_RANK), jnp.float32),
        pltpu.VMEM((N_PER_W,), jnp.float32),
        pltpu.VMEM((N_PER_W,), jnp.float32),
        pltpu.VMEM((LANES,), jnp.float32),
        pltpu.VMEM((N_PER_W,), jnp.float32),
        pltpu.SemaphoreType.DMA,
        pltpu.SemaphoreType.DMA,
    ],
)
def _lfm_sc(uid_hbm, iid_hbm, p_hbm, q_hbm, mu_hbm, ub_hbm, ib_hbm,
            out_hbm, uid_v, iid_v, rows_v, ub_v, ib_v, mu_v, out_v,
            sem, bsem):
    wid = lax.axis_index("s") * NUM_CORES + lax.axis_index("c")
    base = wid * N_PER_W

    pltpu.sync_copy(uid_hbm.at[pl.ds(base, N_PER_W)], uid_v)
    pltpu.sync_copy(iid_hbm.at[pl.ds(base, N_PER_W)], iid_v)
    pltpu.sync_copy(mu_hbm, mu_v)

    cb1 = pltpu.async_copy(ub_hbm.at[uid_v], ub_v, bsem)
    cb2 = pltpu.async_copy(ib_hbm.at[iid_v], ib_v, bsem)

    @pl.loop(0, N_GROUPS)
    def _(g):
        uv = uid_v[pl.ds(g * LANES, LANES)]
        iv = iid_v[pl.ds(g * LANES, LANES)]
        for l in range(LANES):
            j = g * LANES + l
            pltpu.async_copy(p_hbm.at[uv[l]],
                             rows_v.at[j, pl.ds(0, N_RANK)], sem)
            pltpu.async_copy(q_hbm.at[iv[l]],
                             rows_v.at[j, pl.ds(N_RANK, N_RANK)], sem)

    @pl.loop(0, N_PER_W)
    def _(j):
        pltpu.make_async_copy(p_hbm.at[0],
                              rows_v.at[j, pl.ds(0, N_RANK)], sem).wait()
        pltpu.make_async_copy(q_hbm.at[0],
                              rows_v.at[j, pl.ds(N_RANK, N_RANK)],
                              sem).wait()
    cb1.wait()
    cb2.wait()

    mu = mu_v[...]
    lane = lax.iota(jnp.int32, LANES)

    @pl.loop(0, N_GROUPS)
    def _(g):
        rows = g * LANES + lane
        acc = jnp.zeros((LANES,), jnp.float32)
        for d in range(N_RANK):
            u = plsc.load_gather(rows_v, [rows, jnp.full((LANES,), d, jnp.int32)])
            t = plsc.load_gather(rows_v, [rows, jnp.full((LANES,), N_RANK + d, jnp.int32)])
            x = u * t
            acc = acc + jnp.where(x > 0.0, x, SELU_ALPHA * (jnp.exp(x) - 1.0))
        sl = pl.ds(g * LANES, LANES)
        out_v[sl] = SELU_SCALE * acc + ub_v[sl] + ib_v[sl] + mu

    pltpu.sync_copy(out_v, out_hbm.at[pl.ds(base, N_PER_W)])


def kernel(user_ids, item_ids, P, Q, mu, user_bias, item_bias):
    uid = user_ids.astype(jnp.int32)
    iid = item_ids.astype(jnp.int32)
    mu16 = jnp.broadcast_to(mu.astype(jnp.float32), (LANES,))
    return _lfm_sc(uid, iid, P, Q, mu16, user_bias, item_bias)


# speedup vs baseline: 1.8405x; 1.1442x over previous
import functools

import jax
import jax.numpy as jnp
from jax import lax
from jax.experimental import pallas as pl
from jax.experimental.pallas import tpu as pltpu
from jax.experimental.pallas import tpu_sc as plsc

N_RANK = 64
BATCH = 16384

NUM_CORES = 2
NUM_SUBCORES = 16
NUM_WORKERS = NUM_CORES * NUM_SUBCORES
N_PER_W = BATCH // NUM_WORKERS
LANES = 16
N_GROUPS = N_PER_W // LANES
ROW_W = 2 * N_RANK
CHUNK = 4

SELU_ALPHA = 1.6732632423543772
SELU_SCALE = 1.0507009873554805

_mesh = plsc.VectorSubcoreMesh(core_axis_name="c", subcore_axis_name="s")


@functools.partial(
    pl.kernel,
    out_type=jax.ShapeDtypeStruct((BATCH,), jnp.float32),
    mesh=_mesh,
    compiler_params=pltpu.CompilerParams(
        needs_layout_passes=False, use_tc_tiling_on_sc=True),
    scratch_types=[
        pltpu.VMEM((N_PER_W,), jnp.int32),
        pltpu.VMEM((N_PER_W,), jnp.int32),
        pltpu.VMEM((N_PER_W, ROW_W), jnp.float32),
        pltpu.VMEM((CHUNK, N_RANK, 128), jnp.float32),
        pltpu.VMEM((N_PER_W,), jnp.float32),
        pltpu.VMEM((N_PER_W,), jnp.float32),
        pltpu.VMEM((LANES,), jnp.float32),
        pltpu.VMEM((N_PER_W,), jnp.float32),
        pltpu.SemaphoreType.DMA,
        pltpu.SemaphoreType.DMA,
        pltpu.SemaphoreType.DMA,
    ],
)
def _lfm_sc(uid_hbm, iid_hbm, pt_hbm, q_hbm, mu_hbm, ub_hbm, ib_hbm,
            out_hbm, uid_v, iid_v, rows_v, blocks_v, ub_v, ib_v, mu_v,
            out_v, sem, qsem, bsem):
    wid = lax.axis_index("s") * NUM_CORES + lax.axis_index("c")
    base = wid * N_PER_W

    pltpu.sync_copy(uid_hbm.at[pl.ds(base, N_PER_W)], uid_v)
    pltpu.sync_copy(iid_hbm.at[pl.ds(base, N_PER_W)], iid_v)
    pltpu.sync_copy(mu_hbm, mu_v)

    cb1 = pltpu.async_copy(ub_hbm.at[uid_v], ub_v, bsem)
    cb2 = pltpu.async_copy(ib_hbm.at[iid_v], ib_v, bsem)

    @pl.loop(0, N_GROUPS)
    def _(g):
        iv = iid_v[pl.ds(g * LANES, LANES)]
        for l in range(LANES):
            j = g * LANES + l
            pltpu.async_copy(q_hbm.at[iv[l]],
                             rows_v.at[j, pl.ds(N_RANK, N_RANK)], qsem)

    d16 = lax.iota(jnp.int32, LANES)

    @pl.loop(0, N_GROUPS)
    def _(g):
        uv = uid_v[pl.ds(g * LANES, LANES)]
        for sub in range(LANES // CHUNK):
            for l in range(CHUNK):
                s = uv[sub * CHUNK + l]
                cstart = pl.multiple_of((s // 128) * 128, 128)
                pltpu.async_copy(pt_hbm.at[:, pl.ds(cstart, 128)],
                                 blocks_v.at[l], sem)
            for l in range(CHUNK):
                pltpu.make_async_copy(pt_hbm.at[:, pl.ds(0, 128)],
                                      blocks_v.at[l], sem).wait()
            for l in range(CHUNK):
                j = g * LANES + sub * CHUNK + l
                s = uv[sub * CHUNK + l]
                lane16 = jnp.broadcast_to(s % 128, (LANES,))
                slot16 = jnp.full((LANES,), l, jnp.int32)
                j16 = jnp.broadcast_to(j, (LANES,))
                for k in range(N_RANK // LANES):
                    p16 = plsc.load_gather(
                        blocks_v, [slot16, k * LANES + d16, lane16])
                    plsc.store_scatter(rows_v, [j16, k * LANES + d16], p16)

    @pl.loop(0, N_PER_W)
    def _(j):
        pltpu.make_async_copy(q_hbm.at[0],
                              rows_v.at[j, pl.ds(N_RANK, N_RANK)],
                              qsem).wait()
    cb1.wait()
    cb2.wait()

    mu = mu_v[...]
    lane = lax.iota(jnp.int32, LANES)

    @pl.loop(0, N_GROUPS)
    def _(g):
        rows = g * LANES + lane
        acc = jnp.zeros((LANES,), jnp.float32)
        for d in range(N_RANK):
            u = plsc.load_gather(rows_v, [rows, jnp.full((LANES,), d, jnp.int32)])
            t = plsc.load_gather(rows_v, [rows, jnp.full((LANES,), N_RANK + d, jnp.int32)])
            x = u * t
            acc = acc + jnp.where(x > 0.0, x, SELU_ALPHA * (jnp.exp(x) - 1.0))
        sl = pl.ds(g * LANES, LANES)
        out_v[sl] = SELU_SCALE * acc + ub_v[sl] + ib_v[sl] + mu

    pltpu.sync_copy(out_v, out_hbm.at[pl.ds(base, N_PER_W)])


def kernel(user_ids, item_ids, P, Q, mu, user_bias, item_bias):
    uid = user_ids.astype(jnp.int32)
    iid = item_ids.astype(jnp.int32)
    mu16 = jnp.broadcast_to(mu.astype(jnp.float32), (LANES,))
    return _lfm_sc(uid, iid, P.T, Q, mu16, user_bias, item_bias)
